# baseline (speedup 1.0000x reference)
import functools

import jax
import jax.numpy as jnp
from jax import lax
from jax.experimental import pallas as pl
from jax.experimental.pallas import tpu as pltpu
from jax.experimental.pallas import tpu_sc as plsc

_VOCAB = 100000
_BASIC_DIM = 300
_MODIF_DIM = 100
_OUT_DIM = _BASIC_DIM + _MODIF_DIM
_BATCH = 16384

_NC = 2
_NS = 16
_NW = _NC * _NS
_B_PER_W = _BATCH // _NW
_CHUNK = 64
_NCHUNK = _B_PER_W // _CHUNK


def _make_kernel():
    mesh = plsc.VectorSubcoreMesh(core_axis_name="c", subcore_axis_name="s")

    @functools.partial(
        pl.kernel,
        mesh=mesh,
        out_type=jax.ShapeDtypeStruct((_BATCH, _OUT_DIM), jnp.float32),
        compiler_params=pltpu.CompilerParams(
            use_tc_tiling_on_sc=False, needs_layout_passes=False),
        scratch_types=[
            pltpu.VMEM((_B_PER_W,), jnp.int32),
            pltpu.VMEM((_CHUNK,), jnp.int32),
            pltpu.VMEM((_CHUNK,), jnp.int32),
            pltpu.VMEM((_CHUNK, 2 * _BASIC_DIM), jnp.float32),
            pltpu.VMEM((_CHUNK, 2 * _MODIF_DIM), jnp.float32),
            pltpu.VMEM((_CHUNK, _OUT_DIM), jnp.float32),
            pltpu.SemaphoreType.DMA,
        ],
    )
    def k(ids_hbm, basicp_hbm, modifp_hbm, out_hbm,
          idx_v, pidx, parv, buf_p, buf_m, buf_c, sem):
        wid = lax.axis_index("s") * _NC + lax.axis_index("c")
        base = wid * _B_PER_W
        pltpu.sync_copy(ids_hbm.at[pl.ds(base, _B_PER_W)], idx_v)
        iota = lax.iota(jnp.int32, 16)

        def do_chunk(c, carry):
            def prep(t, carry2):
                v = idx_v[pl.ds(c * _CHUNK + t * 16, 16)]
                pidx[pl.ds(t * 16, 16)] = v >> 1
                parv[pl.ds(t * 16, 16)] = v & 1
                return carry2

            lax.fori_loop(0, _CHUNK // 16, prep, 0)

            ga = pltpu.async_copy(basicp_hbm.at[pidx], buf_p, sem)
            gb = pltpu.async_copy(modifp_hbm.at[pidx], buf_m, sem)
            ga.wait()
            gb.wait()

            def interleave(r, carry2):
                rr = jnp.full((16,), r, jnp.int32)
                parb = plsc.load_gather(parv, [rr])
                cb = parb * _BASIC_DIM + iota
                cm = parb * _MODIF_DIM + iota
                for j in range(19):
                    off = 16 * j if j < 18 else _BASIC_DIM - 16
                    v = plsc.load_gather(buf_p, [rr, cb + off])
                    buf_c[r, pl.ds(off, 16)] = v
                for j in range(7):
                    off = 16 * j if j < 6 else _MODIF_DIM - 16
                    v = plsc.load_gather(buf_m, [rr, cm + off])
                    buf_c[r, pl.ds(_BASIC_DIM + off, 16)] = v
                return carry2

            lax.fori_loop(0, _CHUNK, interleave, 0)
            pltpu.sync_copy(
                buf_c, out_hbm.at[pl.ds(base + c * _CHUNK, _CHUNK)])
            return carry

        lax.fori_loop(0, _NCHUNK, do_chunk, 0)

    return k


_kernel_call = _make_kernel()


def kernel(word_ids, basic, modif):
    basicp = basic.reshape(_VOCAB // 2, 2 * _BASIC_DIM)
    modifp = modif.reshape(_VOCAB // 2, 2 * _MODIF_DIM)
    return _kernel_call(word_ids.astype(jnp.int32), basicp, modifp)
